# baseline (speedup 1.0000x reference)
import functools

import jax
import jax.numpy as jnp
from jax import lax
from jax.experimental import pallas as pl
from jax.experimental.pallas import tpu as pltpu
from jax.experimental.pallas import tpu_sc as plsc

_NC = 2
_NS = 16
_NW = _NC * _NS
_SEQ = 128
_NBUF = 2
_INFLIGHT = 1


@functools.lru_cache(maxsize=None)
def _build(b, l, d):
    mesh = plsc.VectorSubcoreMesh(core_axis_name="c", subcore_axis_name="s")

    @functools.partial(
        pl.kernel,
        mesh=mesh,
        out_type=jax.ShapeDtypeStruct((l, b, d), jnp.float32),
        scratch_types=[
            pltpu.VMEM((l, _SEQ), jnp.int32),
            pltpu.VMEM((_NBUF, _SEQ, d), jnp.float32),
            pltpu.SemaphoreType.DMA((_NBUF,)),
            pltpu.SemaphoreType.DMA((_NBUF,)),
        ],
    )
    def emb(idx_hbm, table_hbm, out_hbm, idx_v, rows_v, gsem, ssem):
        wid = lax.axis_index("s") * _NC + lax.axis_index("c")
        sbase = wid * _SEQ
        pltpu.sync_copy(idx_hbm.at[:, pl.ds(sbase, _SEQ)], idx_v)

        def gather(t, start):
            bb = t % _NBUF
            cp = pltpu.make_async_copy(
                table_hbm.at[idx_v.at[t]], rows_v.at[bb], gsem.at[bb]
            )
            cp.start() if start else cp.wait()

        def store(t, start):
            bb = t % _NBUF
            cp = pltpu.make_async_copy(
                rows_v.at[bb], out_hbm.at[t, pl.ds(sbase, _SEQ)], ssem.at[bb]
            )
            cp.start() if start else cp.wait()

        gather(0, start=True)

        def step(t, carry):
            gather(t, start=False)

            @pl.when(t + 1 < l)
            def _():
                gather(t + 1, start=True)

            store(t, start=True)
            store(t, start=False)
            return carry

        lax.fori_loop(0, l, step, 0)

    return emb


def kernel(token_idxs, table):
    b, l = token_idxs.shape
    v, d = table.shape
    idx_t = token_idxs.T.astype(jnp.int32)
    out = _build(b, l, d)(idx_t, table)
    return jnp.transpose(out, (1, 0, 2)).reshape(1, b, l, d)


# speedup vs baseline: 1.2057x; 1.2057x over previous
import functools

import jax
import jax.numpy as jnp
from jax import lax
from jax.experimental import pallas as pl
from jax.experimental.pallas import tpu as pltpu
from jax.experimental.pallas import tpu_sc as plsc

_NC = 2
_NS = 16
_NW = _NC * _NS
_SEQ = 128
_NBUF = 2
_INFLIGHT = 1


@functools.lru_cache(maxsize=None)
def _build(b, l, d):
    mesh = plsc.VectorSubcoreMesh(core_axis_name="c", subcore_axis_name="s")

    @functools.partial(
        pl.kernel,
        mesh=mesh,
        out_type=jax.ShapeDtypeStruct((l, b, d), jnp.float32),
        scratch_types=[
            pltpu.VMEM((l, _SEQ), jnp.int32),
            pltpu.VMEM((_NBUF, _SEQ, d), jnp.float32),
            pltpu.SemaphoreType.DMA((_NBUF,)),
            pltpu.SemaphoreType.DMA((_NBUF,)),
        ],
    )
    def emb(idx_hbm, table_hbm, out_hbm, idx_v, rows_v, gsem, ssem):
        wid = lax.axis_index("s") * _NC + lax.axis_index("c")
        sbase = wid * _SEQ
        pltpu.sync_copy(idx_hbm.at[:, pl.ds(sbase, _SEQ)], idx_v)

        def gather(t, start):
            bb = t % _NBUF
            cp = pltpu.make_async_copy(
                table_hbm.at[idx_v.at[t]], rows_v.at[bb], gsem.at[bb]
            )
            cp.start() if start else cp.wait()

        def store(t, start):
            bb = t % _NBUF
            cp = pltpu.make_async_copy(
                rows_v.at[bb], out_hbm.at[t, pl.ds(sbase, _SEQ)], ssem.at[bb]
            )
            cp.start() if start else cp.wait()

        gather(0, start=True)

        def step(t, carry):
            @pl.when(t + 1 < l)
            def _():
                gather(t + 1, start=True)

            gather(t, start=False)
            store(t, start=True)
            store(t, start=False)
            return carry

        lax.fori_loop(0, l, step, 0)

    return emb


def kernel(token_idxs, table):
    b, l = token_idxs.shape
    v, d = table.shape
    idx_t = token_idxs.T.astype(jnp.int32)
    out = _build(b, l, d)(idx_t, table)
    return jnp.transpose(out, (1, 0, 2)).reshape(1, b, l, d)


# speedup vs baseline: 1.2091x; 1.0028x over previous
import functools

import jax
import jax.numpy as jnp
from jax import lax
from jax.experimental import pallas as pl
from jax.experimental.pallas import tpu as pltpu
from jax.experimental.pallas import tpu_sc as plsc

_NC = 2
_NS = 16
_NW = _NC * _NS
_SEQ = 128
_NBUF = 2


@functools.lru_cache(maxsize=None)
def _build(b, l, d):
    mesh = plsc.VectorSubcoreMesh(core_axis_name="c", subcore_axis_name="s")

    @functools.partial(
        pl.kernel,
        mesh=mesh,
        out_type=jax.ShapeDtypeStruct((l, b, d), jnp.float32),
        scratch_types=[
            pltpu.VMEM((l, _SEQ), jnp.int32),
            pltpu.VMEM((_NBUF, _SEQ, d), jnp.float32),
            pltpu.SemaphoreType.DMA((_NBUF,)),
            pltpu.SemaphoreType.DMA((_NBUF,)),
        ],
    )
    def emb(idx_hbm, table_hbm, out_hbm, idx_v, rows_v, gsem, ssem):
        wid = lax.axis_index("s") * _NC + lax.axis_index("c")
        sbase = wid * _SEQ
        pltpu.sync_copy(idx_hbm.at[:, pl.ds(sbase, _SEQ)], idx_v)

        def gather(t, start):
            bb = t % _NBUF
            cp = pltpu.make_async_copy(
                table_hbm.at[idx_v.at[t]], rows_v.at[bb], gsem.at[bb]
            )
            cp.start() if start else cp.wait()

        def store(t, start):
            bb = t % _NBUF
            cp = pltpu.make_async_copy(
                rows_v.at[bb], out_hbm.at[t, pl.ds(sbase, _SEQ)], ssem.at[bb]
            )
            cp.start() if start else cp.wait()

        gather(0, start=True)

        def step(t, carry):
            @pl.when(t + 1 < l)
            def _():
                gather(t + 1, start=True)

            gather(t, start=False)
            store(t, start=True)
            store(t, start=False)
            return carry

        lax.fori_loop(0, l, step, 0)

    return emb


def kernel(token_idxs, table):
    b, l = token_idxs.shape
    v, d = table.shape
    idx_t = token_idxs.T.astype(jnp.int32)
    out = _build(b, l, d)(idx_t, table)
    return jnp.transpose(out, (1, 0, 2)).reshape(1, b, l, d)


# speedup vs baseline: 1.2440x; 1.0289x over previous
import functools

import jax
import jax.numpy as jnp
from jax import lax
from jax.experimental import pallas as pl
from jax.experimental.pallas import tpu as pltpu
from jax.experimental.pallas import tpu_sc as plsc

_NC = 2
_NS = 16
_NW = _NC * _NS
_SEQ = 128
_NBUF = 2


@functools.lru_cache(maxsize=None)
def _build(b, l, d):
    mesh = plsc.VectorSubcoreMesh(core_axis_name="c", subcore_axis_name="s")

    tpc = 2 if l % 2 == 0 else 1
    n_sl = l // tpc

    @functools.partial(
        pl.kernel,
        mesh=mesh,
        out_type=jax.ShapeDtypeStruct((l, b, d), jnp.float32),
        scratch_types=[
            pltpu.VMEM((l, _SEQ), jnp.int32),
            pltpu.VMEM((_NBUF, tpc, _SEQ, d), jnp.float32),
            pltpu.SemaphoreType.DMA((_NBUF,)),
            pltpu.SemaphoreType.DMA((_NBUF,)),
        ],
    )
    def emb(idx_hbm, table_hbm, out_hbm, idx_v, rows_v, gsem, ssem):
        wid = lax.axis_index("s") * _NC + lax.axis_index("c")
        sbase = wid * _SEQ
        pltpu.sync_copy(idx_hbm.at[:, pl.ds(sbase, _SEQ)], idx_v)

        def gathers(s, start):
            bb = s % _NBUF
            for k in range(tpc):
                cp = pltpu.make_async_copy(
                    table_hbm.at[idx_v.at[s * tpc + k]],
                    rows_v.at[bb, k],
                    gsem.at[bb],
                )
                cp.start() if start else cp.wait()

        def store(s, start):
            bb = s % _NBUF
            cp = pltpu.make_async_copy(
                rows_v.at[bb],
                out_hbm.at[pl.ds(s * tpc, tpc), pl.ds(sbase, _SEQ)],
                ssem.at[bb],
            )
            cp.start() if start else cp.wait()

        gathers(0, start=True)

        def step(s, carry):
            @pl.when(s + 1 < n_sl)
            def _():
                gathers(s + 1, start=True)

            gathers(s, start=False)
            store(s, start=True)
            store(s, start=False)
            return carry

        lax.fori_loop(0, n_sl, step, 0)

    return emb


def kernel(token_idxs, table):
    b, l = token_idxs.shape
    v, d = table.shape
    idx_t = token_idxs.T.astype(jnp.int32)
    out = _build(b, l, d)(idx_t, table)
    return jnp.transpose(out, (1, 0, 2)).reshape(1, b, l, d)
